# baseline (speedup 1.0000x reference)
import functools

import jax
import jax.numpy as jnp
from jax import lax
from jax.experimental import pallas as pl
from jax.experimental.pallas import tpu as pltpu
from jax.experimental.pallas import tpu_sc as plsc

N_TOKENS = 32768
N_EMBD = 4096
N_CHOICES = 64
HIDDEN = N_EMBD // 2

BT = 256

NW = 32
TPW = N_TOKENS // NW
LANES = 16


def _reduce_body(k_ref, x_ref, w1_ref, b1_ref, w2_ref, b2_ref,
                 i1_ref, i2_ref, v1_ref, v2_ref):
    h = jnp.dot(x_ref[...], w1_ref[...], preferred_element_type=jnp.float32)
    h = jnp.maximum(h + b1_ref[...], 0.0)
    logits = jnp.dot(h, w2_ref[...], preferred_element_type=jnp.float32)
    logits = logits + b2_ref[...]

    col = jax.lax.broadcasted_iota(jnp.int32, logits.shape, 1)
    big = jnp.int32(N_CHOICES)
    m1 = jnp.max(logits, axis=-1, keepdims=True)
    i1 = jnp.min(jnp.where(logits == m1, col, big), axis=-1, keepdims=True)
    masked = jnp.where(col == i1, jnp.float32(-jnp.inf), logits)
    m2 = jnp.max(masked, axis=-1, keepdims=True)
    i2 = jnp.min(jnp.where(masked == m2, col, big), axis=-1, keepdims=True)

    p1 = jax.nn.sigmoid(m1 - m2)
    k_is_1 = k_ref[0] == 1
    v1 = jnp.where(k_is_1, jnp.float32(1.0), p1)
    v2 = jnp.where(k_is_1, jnp.float32(0.0), 1.0 - p1)

    i1_ref[...] = i1[:, 0]
    i2_ref[...] = i2[:, 0]
    v1_ref[...] = jnp.broadcast_to(v1, logits.shape[:1] + (1,))[:, 0]
    v2_ref[...] = jnp.broadcast_to(v2, logits.shape[:1] + (1,))[:, 0]


@jax.jit
def _router_reduce(x, W1, b1, W2, b2, k):
    grid = (N_TOKENS // BT,)
    return pl.pallas_call(
        _reduce_body,
        grid=grid,
        in_specs=[
            pl.BlockSpec(memory_space=pltpu.SMEM),
            pl.BlockSpec((BT, N_EMBD), lambda i: (i, 0)),
            pl.BlockSpec((N_EMBD, HIDDEN), lambda i: (0, 0)),
            pl.BlockSpec((1, HIDDEN), lambda i: (0, 0)),
            pl.BlockSpec((HIDDEN, N_CHOICES), lambda i: (0, 0)),
            pl.BlockSpec((1, N_CHOICES), lambda i: (0, 0)),
        ],
        out_specs=[
            pl.BlockSpec((BT,), lambda i: (i,)),
            pl.BlockSpec((BT,), lambda i: (i,)),
            pl.BlockSpec((BT,), lambda i: (i,)),
            pl.BlockSpec((BT,), lambda i: (i,)),
        ],
        out_shape=[
            jax.ShapeDtypeStruct((N_TOKENS,), jnp.int32),
            jax.ShapeDtypeStruct((N_TOKENS,), jnp.int32),
            jax.ShapeDtypeStruct((N_TOKENS,), jnp.float32),
            jax.ShapeDtypeStruct((N_TOKENS,), jnp.float32),
        ],
    )(k, x, W1, b1, W2, b2)


def _scatter_body(i1_hbm, i2_hbm, v1_hbm, v2_hbm, out_hbm,
                  i1v, i2v, v1v, v2v, buf):
    wid = lax.axis_index("s") * 2 + lax.axis_index("c")
    base = wid * TPW

    pltpu.sync_copy(i1_hbm.at[pl.ds(base, TPW)], i1v)
    pltpu.sync_copy(i2_hbm.at[pl.ds(base, TPW)], i2v)
    pltpu.sync_copy(v1_hbm.at[pl.ds(base, TPW)], v1v)
    pltpu.sync_copy(v2_hbm.at[pl.ds(base, TPW)], v2v)

    zero16 = jnp.zeros((LANES,), jnp.float32)

    def _zero_block(i, carry):
        for j in range(16):
            buf[pl.ds(i * (16 * LANES) + j * LANES, LANES)] = zero16
        return carry

    lax.fori_loop(0, TPW * N_CHOICES // (16 * LANES), _zero_block, 0)

    lane = lax.iota(jnp.int32, LANES)
    for g in range(TPW // LANES):
        rowbase = (lane + g * LANES) * N_CHOICES
        i1g = i1v[pl.ds(g * LANES, LANES)]
        i2g = i2v[pl.ds(g * LANES, LANES)]
        v1g = v1v[pl.ds(g * LANES, LANES)]
        v2g = v2v[pl.ds(g * LANES, LANES)]
        plsc.store_scatter(buf, [rowbase + i1g], v1g)
        plsc.store_scatter(buf, [rowbase + i2g], v2g)

    pltpu.sync_copy(buf, out_hbm.at[pl.ds(base * N_CHOICES, TPW * N_CHOICES)])


_scatter_sc = functools.partial(
    pl.kernel,
    out_type=jax.ShapeDtypeStruct((N_TOKENS * N_CHOICES,), jnp.float32),
    mesh=plsc.VectorSubcoreMesh(core_axis_name="c", subcore_axis_name="s"),
    compiler_params=pltpu.CompilerParams(needs_layout_passes=False),
    scratch_types=[
        pltpu.VMEM((TPW,), jnp.int32),
        pltpu.VMEM((TPW,), jnp.int32),
        pltpu.VMEM((TPW,), jnp.float32),
        pltpu.VMEM((TPW,), jnp.float32),
        pltpu.VMEM((TPW * N_CHOICES,), jnp.float32),
    ],
)(_scatter_body)


def kernel(x, W1, b1, W2, b2, k, training):
    k_arr = jnp.asarray(k, jnp.int32).reshape((1,))
    i1, i2, v1, v2 = _router_reduce(
        x, W1, b1.reshape(1, HIDDEN), W2, b2.reshape(1, N_CHOICES), k_arr
    )
    flat = _scatter_sc(i1, i2, v1, v2)
    return flat.reshape(N_TOKENS, N_CHOICES)


# speedup vs baseline: 1.2041x; 1.2041x over previous
import functools

import jax
import jax.numpy as jnp
from jax.experimental import pallas as pl
from jax.experimental.pallas import tpu as pltpu

N_TOKENS = 32768
N_EMBD = 4096
N_CHOICES = 64
HIDDEN = N_EMBD // 2

BT = 256


def _router_body(k_ref, x_ref, w1_ref, b1_ref, w2_ref, b2_ref, o_ref):
    h = jnp.dot(x_ref[...], w1_ref[...], preferred_element_type=jnp.float32)
    h = jnp.maximum(h + b1_ref[...], 0.0)
    logits = jnp.dot(h, w2_ref[...], preferred_element_type=jnp.float32)
    logits = logits + b2_ref[...]

    col = jax.lax.broadcasted_iota(jnp.int32, logits.shape, 1)
    b = jax.lax.bitcast_convert_type(logits, jnp.int32)
    key = b ^ ((b >> 31) & jnp.int32(0x7FFFFFFF))
    key = (key & jnp.int32(~63)) | (jnp.int32(63) - col)
    k1 = jnp.max(key, axis=-1, keepdims=True)
    k2 = jnp.max(
        jnp.where(key == k1, jnp.int32(-0x80000000), key), axis=-1, keepdims=True
    )
    i1 = jnp.int32(63) - (k1 & jnp.int32(63))
    i2 = jnp.int32(63) - (k2 & jnp.int32(63))

    def _unkey(kk):
        ub = kk & jnp.int32(~63)
        return jax.lax.bitcast_convert_type(
            ub ^ ((ub >> 31) & jnp.int32(0x7FFFFFFF)), jnp.float32
        )

    p1 = jax.nn.sigmoid(_unkey(k1) - _unkey(k2))
    k_is_1 = k_ref[0] == 1
    v1 = jnp.where(k_is_1, jnp.float32(1.0), p1)
    v2 = jnp.where(k_is_1, jnp.float32(0.0), 1.0 - p1)
    o_ref[...] = jnp.where(col == i1, v1, jnp.where(col == i2, v2, 0.0))


@functools.partial(jax.jit, static_argnames=("interpret",))
def _router(x, W1, b1, W2, b2, k, interpret=False):
    grid = (N_TOKENS // BT,)
    return pl.pallas_call(
        _router_body,
        grid=grid,
        in_specs=[
            pl.BlockSpec(memory_space=pltpu.SMEM),
            pl.BlockSpec((BT, N_EMBD), lambda i: (i, 0)),
            pl.BlockSpec((N_EMBD, HIDDEN), lambda i: (0, 0)),
            pl.BlockSpec((1, HIDDEN), lambda i: (0, 0)),
            pl.BlockSpec((HIDDEN, N_CHOICES), lambda i: (0, 0)),
            pl.BlockSpec((1, N_CHOICES), lambda i: (0, 0)),
        ],
        out_specs=pl.BlockSpec((BT, N_CHOICES), lambda i: (i, 0)),
        out_shape=jax.ShapeDtypeStruct((N_TOKENS, N_CHOICES), jnp.float32),
        interpret=interpret,
    )(k, x, W1, b1, W2, b2)


def kernel(x, W1, b1, W2, b2, k, training):
    k_arr = jnp.asarray(k, jnp.int32).reshape((1,))
    return _router(
        x, W1, b1.reshape(1, HIDDEN), W2, b2.reshape(1, N_CHOICES), k_arr
    )


# speedup vs baseline: 1.2799x; 1.0630x over previous
import functools

import jax
import jax.numpy as jnp
from jax.experimental import pallas as pl
from jax.experimental.pallas import tpu as pltpu

N_TOKENS = 32768
N_EMBD = 4096
N_CHOICES = 64
HIDDEN = N_EMBD // 2

BT = 512


def _router_body(k_ref, x_ref, w1_ref, b1_ref, w2_ref, b2_ref, o_ref):
    h = jnp.dot(x_ref[...], w1_ref[...], preferred_element_type=jnp.float32)
    h = jnp.maximum(h + b1_ref[...], 0.0)
    logits = jnp.dot(h, w2_ref[...], preferred_element_type=jnp.float32)
    logits = logits + b2_ref[...]

    col = jax.lax.broadcasted_iota(jnp.int32, logits.shape, 1)
    b = jax.lax.bitcast_convert_type(logits, jnp.int32)
    key = b ^ ((b >> 31) & jnp.int32(0x7FFFFFFF))
    key = (key & jnp.int32(~63)) | (jnp.int32(63) - col)
    k1 = jnp.max(key, axis=-1, keepdims=True)
    k2 = jnp.max(
        jnp.where(key == k1, jnp.int32(-0x80000000), key), axis=-1, keepdims=True
    )
    i1 = jnp.int32(63) - (k1 & jnp.int32(63))
    i2 = jnp.int32(63) - (k2 & jnp.int32(63))

    def _unkey(kk):
        ub = kk & jnp.int32(~63)
        return jax.lax.bitcast_convert_type(
            ub ^ ((ub >> 31) & jnp.int32(0x7FFFFFFF)), jnp.float32
        )

    p1 = jax.nn.sigmoid(_unkey(k1) - _unkey(k2))
    k_is_1 = k_ref[0] == 1
    v1 = jnp.where(k_is_1, jnp.float32(1.0), p1)
    v2 = jnp.where(k_is_1, jnp.float32(0.0), 1.0 - p1)
    o_ref[...] = jnp.where(col == i1, v1, jnp.where(col == i2, v2, 0.0))


@functools.partial(jax.jit, static_argnames=("interpret",))
def _router(x, W1, b1, W2, b2, k, interpret=False):
    grid = (N_TOKENS // BT,)
    return pl.pallas_call(
        _router_body,
        grid=grid,
        in_specs=[
            pl.BlockSpec(memory_space=pltpu.SMEM),
            pl.BlockSpec((BT, N_EMBD), lambda i: (i, 0)),
            pl.BlockSpec((N_EMBD, HIDDEN), lambda i: (0, 0)),
            pl.BlockSpec((1, HIDDEN), lambda i: (0, 0)),
            pl.BlockSpec((HIDDEN, N_CHOICES), lambda i: (0, 0)),
            pl.BlockSpec((1, N_CHOICES), lambda i: (0, 0)),
        ],
        out_specs=pl.BlockSpec((BT, N_CHOICES), lambda i: (i, 0)),
        out_shape=jax.ShapeDtypeStruct((N_TOKENS, N_CHOICES), jnp.float32),
        compiler_params=pltpu.CompilerParams(vmem_limit_bytes=100 * 1024 * 1024),
        interpret=interpret,
    )(k, x, W1, b1, W2, b2)


def kernel(x, W1, b1, W2, b2, k, training):
    k_arr = jnp.asarray(k, jnp.int32).reshape((1,))
    return _router(
        x, W1, b1.reshape(1, HIDDEN), W2, b2.reshape(1, N_CHOICES), k_arr
    )
